# baseline (speedup 1.0000x reference)
import functools

import jax
import jax.numpy as jnp
from jax import lax
from jax.experimental import pallas as pl
from jax.experimental.pallas import tpu as pltpu
from jax.experimental.pallas import tpu_sc as plsc

DIAM = 512
EDIM = 64
TSTRIDE = 65


def _renorm_body(t_ref, o_ref):
    t = t_ref[...]
    norms = jnp.max(jnp.abs(t), axis=1, keepdims=True)
    scale = jnp.where(norms > 1.0, 1.0 / (norms + 1e-7), 1.0)
    o_ref[...] = t * scale


def _renorm(table):
    return pl.pallas_call(
        _renorm_body,
        out_shape=jax.ShapeDtypeStruct(table.shape, table.dtype),
    )(table)


def _sc_gather_t(x, table):
    NB_, T = x.shape
    V = table.shape[0]
    NW = 32
    BPW = NB_ // NW
    NB = 2
    n_tg = T // NB
    assert BPW * NW == NB_ and n_tg * NB == T

    mesh = plsc.VectorSubcoreMesh(core_axis_name="c", subcore_axis_name="s")

    @functools.partial(
        pl.kernel,
        mesh=mesh,
        compiler_params=pltpu.CompilerParams(needs_layout_passes=False),
        out_type=jax.ShapeDtypeStruct((T, EDIM, NB_), jnp.float32),
        scratch_types=[
            pltpu.VMEM((64, EDIM), jnp.float32),
            pltpu.VMEM((V * TSTRIDE,), jnp.float32),
            pltpu.VMEM((BPW, T), jnp.int32),
            pltpu.VMEM((T, BPW), jnp.int32),
            pltpu.VMEM((NB, EDIM, BPW), jnp.float32),
            pltpu.SemaphoreType.DMA,
            pltpu.SemaphoreType.DMA,
            pltpu.SemaphoreType.DMA((NB,)),
        ],
    )
    def k(x_hbm, tbl_hbm, out_hbm, stg_v, tbl_v, idx_v, idxt_v, tiles_v,
          sem_t, sem_i, sem_s):
        wid = lax.axis_index("s") * 2 + lax.axis_index("c")
        b0 = wid * BPW
        lanes = lax.broadcasted_iota(jnp.int32, (16,), 0)


        for s in range(9):
            r0, nr = (s * 64, 64) if s < 8 else (512, 1)
            copy_t = pltpu.make_async_copy(
                tbl_hbm.at[pl.ds(r0, nr)], stg_v.at[pl.ds(0, nr)], sem_t)
            copy_t.start()
            copy_t.wait()

            def repack(r, c):
                base = (r0 + r) * TSTRIDE
                for kk in range(EDIM // 16):
                    v = stg_v[r, pl.ds(kk * 16, 16)]
                    plsc.store_scatter(
                        tbl_v, [base + kk * 16 + lanes], v)
                return c

            lax.fori_loop(0, nr, repack, 0)

        copy_i = pltpu.make_async_copy(x_hbm.at[pl.ds(b0, BPW)], idx_v, sem_i)
        copy_i.start()
        copy_i.wait()

        def tr_row(t, c):
            tvec = jnp.full((16,), t, dtype=jnp.int32)
            for g in range(BPW // 16):
                v = plsc.load_gather(idx_v, [g * 16 + lanes, tvec])
                idxt_v[t, pl.ds(g * 16, 16)] = v
            return c

        lax.fori_loop(0, T, tr_row, 0)

        def wait_store(nb):
            pltpu.make_async_copy(
                tiles_v.at[nb], out_hbm.at[0, :, pl.ds(0, BPW)],
                sem_s.at[nb]).wait()

        def fill_tile(t, nb):
            @plsc.parallel_loop(0, BPW // 16, unroll=8)
            def grp(g):
                iv = jnp.minimum(idxt_v[t, pl.ds(g * 16, 16)], DIAM)
                a = iv * TSTRIDE
                for d in range(EDIM):
                    v = plsc.load_gather(tbl_v, [a + d])
                    tiles_v[nb, d, pl.ds(g * 16, 16)] = v

        def tgroup(tg, c):
            t0 = tg * NB
            for nb in range(NB):
                t = t0 + nb

                @pl.when(t >= NB)
                def _():
                    wait_store(nb)

                fill_tile(t, nb)
                pltpu.make_async_copy(
                    tiles_v.at[nb], out_hbm.at[t, :, pl.ds(b0, BPW)],
                    sem_s.at[nb]).start()
            return c

        lax.fori_loop(0, n_tg, tgroup, 0)

        for nb in range(NB):
            wait_store(nb)

    return k(x, table)


def kernel(x, table):
    renormed = _renorm(table)
    out_t = _sc_gather_t(x, renormed)
    return jnp.transpose(out_t, (2, 0, 1))


# speedup vs baseline: 1.5860x; 1.5860x over previous
import functools

import jax
import jax.numpy as jnp
from jax import lax
from jax.experimental import pallas as pl
from jax.experimental.pallas import tpu as pltpu
from jax.experimental.pallas import tpu_sc as plsc

DIAM = 512
EDIM = 64
TSTRIDE = 65


def _renorm_body(t_ref, o_ref):
    t = t_ref[...]
    norms = jnp.max(jnp.abs(t), axis=1, keepdims=True)
    scale = jnp.where(norms > 1.0, 1.0 / (norms + 1e-7), 1.0)
    o_ref[...] = t * scale


def _renorm(table):
    return pl.pallas_call(
        _renorm_body,
        out_shape=jax.ShapeDtypeStruct(table.shape, table.dtype),
    )(table)


def _sc_gather_t(x, table):
    NB_, T = x.shape
    V = table.shape[0]
    NW = 32
    BPW = NB_ // NW
    NB = 2
    n_tg = T // NB
    assert BPW * NW == NB_ and n_tg * NB == T

    mesh = plsc.VectorSubcoreMesh(core_axis_name="c", subcore_axis_name="s")

    @functools.partial(
        pl.kernel,
        mesh=mesh,
        compiler_params=pltpu.CompilerParams(needs_layout_passes=False),
        out_type=jax.ShapeDtypeStruct((T, EDIM, NB_), jnp.float32),
        scratch_types=[
            pltpu.VMEM((64, EDIM), jnp.float32),
            pltpu.VMEM((V * TSTRIDE,), jnp.float32),
            pltpu.VMEM((BPW, T), jnp.int32),
            pltpu.VMEM((T, BPW), jnp.int32),
            pltpu.VMEM((NB, EDIM, BPW), jnp.float32),
            pltpu.SemaphoreType.DMA,
            pltpu.SemaphoreType.DMA,
            pltpu.SemaphoreType.DMA((NB,)),
        ],
    )
    def k(x_hbm, tbl_hbm, out_hbm, stg_v, tbl_v, idx_v, idxt_v, tiles_v,
          sem_t, sem_i, sem_s):
        wid = lax.axis_index("s") * 2 + lax.axis_index("c")
        b0 = wid * BPW
        lanes = lax.broadcasted_iota(jnp.int32, (16,), 0)


        for s in range(9):
            r0, nr = (s * 64, 64) if s < 8 else (512, 1)
            copy_t = pltpu.make_async_copy(
                tbl_hbm.at[pl.ds(r0, nr)], stg_v.at[pl.ds(0, nr)], sem_t)
            copy_t.start()
            copy_t.wait()

            def repack(r, c):
                base = (r0 + r) * TSTRIDE
                for kk in range(EDIM // 16):
                    v = stg_v[r, pl.ds(kk * 16, 16)]
                    plsc.store_scatter(
                        tbl_v, [base + kk * 16 + lanes], v)
                return c

            lax.fori_loop(0, nr, repack, 0)

        copy_i = pltpu.make_async_copy(x_hbm.at[pl.ds(b0, BPW)], idx_v, sem_i)
        copy_i.start()
        copy_i.wait()

        @plsc.parallel_loop(0, T, unroll=4)
        def tr_row(t):
            tvec = jnp.full((16,), t, dtype=jnp.int32)
            for g in range(BPW // 16):
                v = plsc.load_gather(idx_v, [g * 16 + lanes, tvec])
                idxt_v[t, pl.ds(g * 16, 16)] = v

        def wait_store(nb):
            pltpu.make_async_copy(
                tiles_v.at[nb], out_hbm.at[0, :, pl.ds(0, BPW)],
                sem_s.at[nb]).wait()

        def fill_tile(t, nb):
            @plsc.parallel_loop(0, BPW // 16, unroll=4)
            def grp(g):
                iv = jnp.minimum(idxt_v[t, pl.ds(g * 16, 16)], DIAM)
                a = iv * TSTRIDE
                for d in range(EDIM):
                    v = plsc.load_gather(tbl_v, [a + d])
                    tiles_v[nb, d, pl.ds(g * 16, 16)] = v

        def tgroup(tg, c):
            t0 = tg * NB
            for nb in range(NB):
                t = t0 + nb

                @pl.when(t >= NB)
                def _():
                    wait_store(nb)

                fill_tile(t, nb)
                pltpu.make_async_copy(
                    tiles_v.at[nb], out_hbm.at[t, :, pl.ds(b0, BPW)],
                    sem_s.at[nb]).start()
            return c

        lax.fori_loop(0, n_tg, tgroup, 0)

        for nb in range(NB):
            wait_store(nb)

    return k(x, table)


def kernel(x, table):
    renormed = _renorm(table)
    out_t = _sc_gather_t(x, renormed)
    return jnp.transpose(out_t, (2, 0, 1))


# speedup vs baseline: 1.6529x; 1.0422x over previous
import functools

import jax
import jax.numpy as jnp
from jax import lax
from jax.experimental import pallas as pl
from jax.experimental.pallas import tpu as pltpu
from jax.experimental.pallas import tpu_sc as plsc

DIAM = 512
EDIM = 64
TSTRIDE = 65


def _renorm_body(t_ref, o_ref):
    t = t_ref[...]
    norms = jnp.max(jnp.abs(t), axis=1, keepdims=True)
    scale = jnp.where(norms > 1.0, 1.0 / (norms + 1e-7), 1.0)
    o_ref[...] = t * scale


def _renorm(table):
    return pl.pallas_call(
        _renorm_body,
        out_shape=jax.ShapeDtypeStruct(table.shape, table.dtype),
    )(table)


def _sc_gather_t(x, table):
    NB_, T = x.shape
    V = table.shape[0]
    NW = 32
    BPW = NB_ // NW
    NB = 2
    n_tg = T // NB
    assert BPW * NW == NB_ and n_tg * NB == T

    mesh = plsc.VectorSubcoreMesh(core_axis_name="c", subcore_axis_name="s")

    @functools.partial(
        pl.kernel,
        mesh=mesh,
        compiler_params=pltpu.CompilerParams(needs_layout_passes=False),
        out_type=jax.ShapeDtypeStruct((T, EDIM, NB_), jnp.float32),
        scratch_types=[
            pltpu.VMEM((2, 64, EDIM), jnp.float32),
            pltpu.VMEM((V * TSTRIDE,), jnp.float32),
            pltpu.VMEM((BPW, T), jnp.int32),
            pltpu.VMEM((T, BPW), jnp.int32),
            pltpu.VMEM((NB, EDIM, BPW), jnp.float32),
            pltpu.SemaphoreType.DMA((2,)),
            pltpu.SemaphoreType.DMA,
            pltpu.SemaphoreType.DMA((NB,)),
        ],
    )
    def k(x_hbm, tbl_hbm, out_hbm, stg_v, tbl_v, idx_v, idxt_v, tiles_v,
          sem_t, sem_i, sem_s):
        wid = lax.axis_index("s") * 2 + lax.axis_index("c")
        b0 = wid * BPW
        lanes = lax.broadcasted_iota(jnp.int32, (16,), 0)


        copy_i = pltpu.make_async_copy(x_hbm.at[pl.ds(b0, BPW)], idx_v, sem_i)
        copy_i.start()

        def stage_copy(s):
            r0, nr = (s * 64, 64) if s < 8 else (512, 1)
            return pltpu.make_async_copy(
                tbl_hbm.at[pl.ds(r0, nr)],
                stg_v.at[s % 2].at[pl.ds(0, nr)], sem_t.at[s % 2])

        stage_copy(0).start()
        for s in range(9):
            if s < 8:
                stage_copy(s + 1).start()
            stage_copy(s).wait()
            r0, nr = (s * 64, 64) if s < 8 else (512, 1)
            sb = s % 2

            @plsc.parallel_loop(0, nr, unroll=4 if nr > 1 else 1)
            def repack(r):
                base = (r0 + r) * TSTRIDE
                for kk in range(EDIM // 16):
                    v = stg_v[sb, r, pl.ds(kk * 16, 16)]
                    plsc.store_scatter(
                        tbl_v, [base + kk * 16 + lanes], v)

        copy_i.wait()

        @plsc.parallel_loop(0, T, unroll=4)
        def tr_row(t):
            tvec = jnp.full((16,), t, dtype=jnp.int32)
            for g in range(BPW // 16):
                v = plsc.load_gather(idx_v, [g * 16 + lanes, tvec])
                idxt_v[t, pl.ds(g * 16, 16)] = v

        def wait_store(nb):
            pltpu.make_async_copy(
                tiles_v.at[nb], out_hbm.at[0, :, pl.ds(0, BPW)],
                sem_s.at[nb]).wait()

        def fill_tile(t, nb):
            @plsc.parallel_loop(0, BPW // 16, unroll=4)
            def grp(g):
                iv = jnp.minimum(idxt_v[t, pl.ds(g * 16, 16)], DIAM)
                a = iv * TSTRIDE
                for d in range(EDIM):
                    v = plsc.load_gather(tbl_v, [a + d])
                    tiles_v[nb, d, pl.ds(g * 16, 16)] = v

        def tgroup(tg, c):
            t0 = tg * NB
            for nb in range(NB):
                t = t0 + nb

                @pl.when(t >= NB)
                def _():
                    wait_store(nb)

                fill_tile(t, nb)
                pltpu.make_async_copy(
                    tiles_v.at[nb], out_hbm.at[t, :, pl.ds(b0, BPW)],
                    sem_s.at[nb]).start()
            return c

        lax.fori_loop(0, n_tg, tgroup, 0)

        for nb in range(NB):
            wait_store(nb)

    return k(x, table)


def kernel(x, table):
    renormed = _renorm(table)
    out_t = _sc_gather_t(x, renormed)
    return jnp.transpose(out_t, (2, 0, 1))
